# Initial kernel scaffold; baseline (speedup 1.0000x reference)
import jax
import jax.numpy as jnp
from jax.experimental import pallas as pl


def kernel(tokens0, tokens1, tokens2, params):
    raise NotImplementedError("write your pallas kernel here")


# speedup vs baseline: 1.0408x; 1.0408x over previous
import functools

import jax
import jax.numpy as jnp
from jax.experimental import pallas as pl
from jax.experimental.pallas import tpu as pltpu

DIMC = 1024
HEADSC = 16
DHC = 64
BLKC = 32
SELC = 8
FFH = 4096
ROWT = 256


def _rms(x, w):
    return x * jax.lax.rsqrt(jnp.mean(x * x, axis=-1, keepdims=True) + 1e-6) * w


def _qkvg_body(x_ref, nw_ref, w_ref, o_ref):
    xn = _rms(x_ref[...], nw_ref[...])
    o_ref[...] = jnp.dot(xn, w_ref[...], preferred_element_type=jnp.float32)


def _oproj_body(a_ref, w_ref, r_ref, o_ref):
    o_ref[...] = jnp.dot(a_ref[...], w_ref[...],
                         preferred_element_type=jnp.float32) + r_ref[...]


def _ff1_body(x_ref, nw_ref, w1_ref, b1_ref, o_ref):
    xn = _rms(x_ref[...], nw_ref[...])
    h = jnp.dot(xn, w1_ref[...], preferred_element_type=jnp.float32) + b1_ref[...]
    o_ref[...] = jax.nn.gelu(h)


def _ff2_body(h_ref, w2_ref, b2_ref, r_ref, o_ref):
    o_ref[...] = (jnp.dot(h_ref[...], w2_ref[...],
                          preferred_element_type=jnp.float32)
                  + b2_ref[...] + r_ref[...])


def _rows_mm(body, x, aux_full, out_cols, extra_row=None):
    n, _ = x.shape
    grid = (n // ROWT,)
    in_specs = [pl.BlockSpec((ROWT, x.shape[1]), lambda i: (i, 0))]
    args = [x]
    for a in aux_full:
        in_specs.append(pl.BlockSpec(a.shape, lambda i, r=a.ndim: (0,) * r))
        args.append(a)
    if extra_row is not None:
        in_specs.append(pl.BlockSpec((ROWT, extra_row.shape[1]), lambda i: (i, 0)))
        args.append(extra_row)
    return pl.pallas_call(
        body,
        grid=grid,
        in_specs=in_specs,
        out_specs=pl.BlockSpec((ROWT, out_cols), lambda i: (i, 0)),
        out_shape=jax.ShapeDtypeStruct((n, out_cols), jnp.float32),
    )(*args)


def _pool_body(k_ref, v_ref, wk_ref, wv_ref, ck_ref, cv_ref):
    s = k_ref.shape[2]
    nb = s // BLKC
    for src, wp, dst in ((k_ref, wk_ref, ck_ref), (v_ref, wv_ref, cv_ref)):
        blocks = src[0, 0].reshape(nb, BLKC, DHC)
        logits = jax.lax.dot_general(
            src[0, 0], wp[...], (((1,), (0,)), ((), ())),
            preferred_element_type=jnp.float32).reshape(nb, BLKC, DHC)
        attn = jax.nn.softmax(logits, axis=-2)
        dst[0, 0] = jnp.sum(blocks * attn, axis=-2)


def _pool(k, v, wp_k, wp_v):
    b, h, s, _ = k.shape
    nb = s // BLKC
    spec_s = pl.BlockSpec((1, 1, s, DHC), lambda i, j: (i, j, 0, 0))
    spec_w = pl.BlockSpec((DHC, DHC), lambda i, j: (0, 0))
    spec_o = pl.BlockSpec((1, 1, nb, DHC), lambda i, j: (i, j, 0, 0))
    return pl.pallas_call(
        _pool_body,
        grid=(b, h),
        in_specs=[spec_s, spec_s, spec_w, spec_w],
        out_specs=[spec_o, spec_o],
        out_shape=[jax.ShapeDtypeStruct((b, h, nb, DHC), jnp.float32)] * 2,
    )(k, v, wp_k, wp_v)


def _coarse_body(q_ref, ck_ref, cv_ref, oc_ref, idx_ref):
    s = q_ref.shape[2]
    nb = ck_ref.shape[2]
    nqb = s // BLKC
    scale = DHC ** -0.5
    sc = jax.lax.dot_general(q_ref[0, 0], ck_ref[0, 0], (((1,), (1,)), ((), ())),
                             preferred_element_type=jnp.float32) * scale
    attn = jax.nn.softmax(sc, axis=-1)
    oc_ref[0, 0] = jnp.dot(attn, cv_ref[0, 0], preferred_element_type=jnp.float32)
    imp = jnp.mean(attn.reshape(nqb, BLKC, nb), axis=1)
    lane = jax.lax.broadcasted_iota(jnp.int32, (nqb, nb), 1)
    cols = []
    for _ in range(SELC):
        best = jnp.argmax(imp, axis=1).astype(jnp.int32)
        cols.append(best)
        imp = jnp.where(lane == best[:, None], -jnp.inf, imp)
    idx_ref[0, 0] = jnp.stack(cols, axis=1)


def _coarse(q, ck, cv):
    b, h, s, _ = q.shape
    nb = ck.shape[2]
    nqb = s // BLKC
    return pl.pallas_call(
        _coarse_body,
        grid=(b, h),
        in_specs=[pl.BlockSpec((1, 1, s, DHC), lambda i, j: (i, j, 0, 0)),
                  pl.BlockSpec((1, 1, nb, DHC), lambda i, j: (i, j, 0, 0)),
                  pl.BlockSpec((1, 1, nb, DHC), lambda i, j: (i, j, 0, 0))],
        out_specs=[pl.BlockSpec((1, 1, s, DHC), lambda i, j: (i, j, 0, 0)),
                   pl.BlockSpec((1, 1, nqb, SELC), lambda i, j: (i, j, 0, 0))],
        out_shape=[jax.ShapeDtypeStruct((b, h, s, DHC), jnp.float32),
                   jax.ShapeDtypeStruct((b, h, nqb, SELC), jnp.int32)],
    )(q, ck, cv)


def _fine_body(idx_ref, q_ref, k_ref, v_ref, oc_ref, g_ref, o_ref):
    b = pl.program_id(0)
    h = pl.program_id(1)
    s = q_ref.shape[2]
    nqb = s // BLKC
    scale = DHC ** -0.5

    def body(i, _):
        base = ((b * HEADSC + h) * nqb + i) * SELC
        qb = q_ref[0, 0, pl.ds(i * BLKC, BLKC), :]
        fk = jnp.concatenate(
            [k_ref[0, 0, pl.ds(idx_ref[base + j] * BLKC, BLKC), :]
             for j in range(SELC)], axis=0)
        fv = jnp.concatenate(
            [v_ref[0, 0, pl.ds(idx_ref[base + j] * BLKC, BLKC), :]
             for j in range(SELC)], axis=0)
        sf = jax.lax.dot_general(qb, fk, (((1,), (1,)), ((), ())),
                                 preferred_element_type=jnp.float32) * scale
        attn = jax.nn.softmax(sf, axis=-1)
        of = jnp.dot(attn, fv, preferred_element_type=jnp.float32)
        g = jax.nn.sigmoid(g_ref[0, 0, pl.ds(i * BLKC, BLKC), :])
        oc = oc_ref[0, 0, pl.ds(i * BLKC, BLKC), :]
        o_ref[0, 0, pl.ds(i * BLKC, BLKC), :] = g[:, 0:1] * oc + g[:, 1:2] * of
        return 0

    jax.lax.fori_loop(0, nqb, body, 0)


def _fine(idx, q, k, v, oc, g):
    b, h, s, _ = q.shape
    spec_s = pl.BlockSpec((1, 1, s, DHC), lambda i, j, *_: (i, j, 0, 0))
    spec_g = pl.BlockSpec((1, 1, s, 2), lambda i, j, *_: (i, j, 0, 0))
    grid_spec = pltpu.PrefetchScalarGridSpec(
        num_scalar_prefetch=1,
        grid=(b, h),
        in_specs=[spec_s, spec_s, spec_s, spec_s, spec_g],
        out_specs=spec_s,
    )
    return pl.pallas_call(
        _fine_body,
        grid_spec=grid_spec,
        out_shape=jax.ShapeDtypeStruct((b, h, s, DHC), jnp.float32),
    )(idx.reshape(-1), q, k, v, oc, g)


def _fnorm_body(x_ref, nw_ref, o_ref):
    o_ref[...] = _rms(x_ref[...], nw_ref[...])


def _sparse_attn(x, p):
    b, s, _ = x.shape
    rows = b * s
    x2 = x.reshape(rows, DIMC)
    wcat = jnp.concatenate([p['wq'], p['wk'], p['wv'], p['wg']], axis=1)
    y = _rows_mm(_qkvg_body, x2, [p['norm_w'].reshape(1, DIMC), wcat],
                 3 * DIMC + 2 * HEADSC)
    q = y[:, :DIMC].reshape(b, s, HEADSC, DHC).transpose(0, 2, 1, 3)
    k = y[:, DIMC:2 * DIMC].reshape(b, s, HEADSC, DHC).transpose(0, 2, 1, 3)
    v = y[:, 2 * DIMC:3 * DIMC].reshape(b, s, HEADSC, DHC).transpose(0, 2, 1, 3)
    g = y[:, 3 * DIMC:].reshape(b, s, HEADSC, 2).transpose(0, 2, 1, 3)
    ck, cv = _pool(k, v, p['wp_k'], p['wp_v'])
    oc, idx = _coarse(q, ck, cv)
    o = _fine(idx, q, k, v, oc, g)
    o2 = o.transpose(0, 2, 1, 3).reshape(rows, HEADSC * DHC)
    out = _rows_mm(_oproj_body, o2, [p['wo']], DIMC, extra_row=x2)
    return out.reshape(b, s, DIMC)


def _ffl(x, p):
    b, s, _ = x.shape
    rows = b * s
    x2 = x.reshape(rows, DIMC)
    h = _rows_mm(_ff1_body, x2,
                 [p['ffnorm_w'].reshape(1, DIMC), p['w1'], p['b1'].reshape(1, FFH)],
                 FFH)
    out = _rows_mm(_ff2_body, h, [p['w2'], p['b2'].reshape(1, DIMC)], DIMC,
                   extra_row=x2)
    return out.reshape(b, s, DIMC)


def kernel(tokens0, tokens1, tokens2, params):
    outs = []
    for i, tokens in enumerate((tokens0, tokens1, tokens2)):
        t = tokens
        for layer in params['levels'][i]:
            t = _sparse_attn(t, layer)
            t = _ffl(t, layer)
        b, s, _ = t.shape
        t2 = t.reshape(b * s, DIMC)
        o = _rows_mm(_fnorm_body, t2,
                     [params['final_norm'].reshape(1, DIMC)], DIMC)
        outs.append(o.reshape(b, s, DIMC))
    return tuple(outs)


# speedup vs baseline: 1.1063x; 1.0629x over previous
import functools

import jax
import jax.numpy as jnp
from jax.experimental import pallas as pl
from jax.experimental.pallas import tpu as pltpu

DIMC = 1024
HEADSC = 16
DHC = 64
BLKC = 32
SELC = 8
FFH = 4096
ROWT = 256


def _rms(x, w):
    return x * jax.lax.rsqrt(jnp.mean(x * x, axis=-1, keepdims=True) + 1e-6) * w


def _qkvg_body(x_ref, nw_ref, w_ref, o_ref):
    xn = _rms(x_ref[...], nw_ref[...])
    o_ref[...] = jnp.dot(xn, w_ref[...], preferred_element_type=jnp.float32)


def _oproj_body(a_ref, w_ref, r_ref, o_ref):
    o_ref[...] = jnp.dot(a_ref[...], w_ref[...],
                         preferred_element_type=jnp.float32) + r_ref[...]


def _ff1_body(x_ref, nw_ref, w1_ref, b1_ref, o_ref):
    xn = _rms(x_ref[...], nw_ref[...])
    h = jnp.dot(xn, w1_ref[...], preferred_element_type=jnp.float32) + b1_ref[...]
    o_ref[...] = jax.nn.gelu(h)


def _ff2_body(h_ref, w2_ref, b2_ref, r_ref, o_ref):
    o_ref[...] = (jnp.dot(h_ref[...], w2_ref[...],
                          preferred_element_type=jnp.float32)
                  + b2_ref[...] + r_ref[...])


def _rows_mm(body, x, aux_full, out_cols, extra_row=None):
    n, _ = x.shape
    grid = (n // ROWT,)
    in_specs = [pl.BlockSpec((ROWT, x.shape[1]), lambda i: (i, 0))]
    args = [x]
    for a in aux_full:
        in_specs.append(pl.BlockSpec(a.shape, lambda i, r=a.ndim: (0,) * r))
        args.append(a)
    if extra_row is not None:
        in_specs.append(pl.BlockSpec((ROWT, extra_row.shape[1]), lambda i: (i, 0)))
        args.append(extra_row)
    return pl.pallas_call(
        body,
        grid=grid,
        in_specs=in_specs,
        out_specs=pl.BlockSpec((ROWT, out_cols), lambda i: (i, 0)),
        out_shape=jax.ShapeDtypeStruct((n, out_cols), jnp.float32),
    )(*args)


def _pool_body(k_ref, v_ref, wk_ref, wv_ref, ck_ref, cv_ref):
    s = k_ref.shape[2]
    nb = s // BLKC
    for src, wp, dst in ((k_ref, wk_ref, ck_ref), (v_ref, wv_ref, cv_ref)):
        blocks = src[0, 0].reshape(nb, BLKC, DHC)
        logits = jax.lax.dot_general(
            src[0, 0], wp[...], (((1,), (0,)), ((), ())),
            preferred_element_type=jnp.float32).reshape(nb, BLKC, DHC)
        attn = jax.nn.softmax(logits, axis=-2)
        dst[0, 0] = jnp.sum(blocks * attn, axis=-2)


def _pool(k, v, wp_k, wp_v):
    b, h, s, _ = k.shape
    nb = s // BLKC
    spec_s = pl.BlockSpec((1, 1, s, DHC), lambda i, j: (i, j, 0, 0))
    spec_w = pl.BlockSpec((DHC, DHC), lambda i, j: (0, 0))
    spec_o = pl.BlockSpec((1, 1, nb, DHC), lambda i, j: (i, j, 0, 0))
    return pl.pallas_call(
        _pool_body,
        grid=(b, h),
        in_specs=[spec_s, spec_s, spec_w, spec_w],
        out_specs=[spec_o, spec_o],
        out_shape=[jax.ShapeDtypeStruct((b, h, nb, DHC), jnp.float32)] * 2,
    )(k, v, wp_k, wp_v)


def _coarse_body(q_ref, ck_ref, cv_ref, oc_ref, idx_ref):
    s = q_ref.shape[2]
    nb = ck_ref.shape[2]
    nqb = s // BLKC
    scale = DHC ** -0.5
    sc = jax.lax.dot_general(q_ref[0, 0], ck_ref[0, 0], (((1,), (1,)), ((), ())),
                             preferred_element_type=jnp.float32) * scale
    attn = jax.nn.softmax(sc, axis=-1)
    oc_ref[0, 0] = jnp.dot(attn, cv_ref[0, 0], preferred_element_type=jnp.float32)
    imp = jnp.mean(attn.reshape(nqb, BLKC, nb), axis=1)
    lane = jax.lax.broadcasted_iota(jnp.int32, (nqb, nb), 1)
    cols = []
    for _ in range(SELC):
        best = jnp.argmax(imp, axis=1).astype(jnp.int32)
        cols.append(best)
        imp = jnp.where(lane == best[:, None], -jnp.inf, imp)
    idx_ref[0, 0] = jnp.stack(cols, axis=1)


def _coarse(q, ck, cv):
    b, h, s, _ = q.shape
    nb = ck.shape[2]
    nqb = s // BLKC
    return pl.pallas_call(
        _coarse_body,
        grid=(b, h),
        in_specs=[pl.BlockSpec((1, 1, s, DHC), lambda i, j: (i, j, 0, 0)),
                  pl.BlockSpec((1, 1, nb, DHC), lambda i, j: (i, j, 0, 0)),
                  pl.BlockSpec((1, 1, nb, DHC), lambda i, j: (i, j, 0, 0))],
        out_specs=[pl.BlockSpec((1, 1, s, DHC), lambda i, j: (i, j, 0, 0)),
                   pl.BlockSpec((1, 1, nqb, SELC), lambda i, j: (i, j, 0, 0))],
        out_shape=[jax.ShapeDtypeStruct((b, h, s, DHC), jnp.float32),
                   jax.ShapeDtypeStruct((b, h, nqb, SELC), jnp.int32)],
    )(q, ck, cv)


def _gather_body(idx_ref, k2_ref, v2_ref, fk_ref, fv_ref):
    nb = k2_ref.shape[2]
    nqb = idx_ref.shape[2]
    idx = idx_ref[0, 0]
    k2 = k2_ref[0, 0]
    v2 = v2_ref[0, 0]
    lane = jax.lax.broadcasted_iota(jnp.int32, (nqb, nb), 1)
    for j in range(SELC):
        oh = (lane == idx[:, j:j + 1]).astype(jnp.float32)
        fk_ref[0, 0, :, j, :] = jnp.dot(
            oh, k2, preferred_element_type=jnp.float32).astype(jnp.bfloat16)
        fv_ref[0, 0, :, j, :] = jnp.dot(
            oh, v2, preferred_element_type=jnp.float32).astype(jnp.bfloat16)


def _gather(idx, k2, v2):
    b, h, nb, _ = k2.shape
    nqb = idx.shape[2]
    spec_i = pl.BlockSpec((1, 1, nqb, SELC), lambda i, j: (i, j, 0, 0))
    spec_k = pl.BlockSpec((1, 1, nb, BLKC * DHC), lambda i, j: (i, j, 0, 0))
    spec_o = pl.BlockSpec((1, 1, nqb, SELC, BLKC * DHC),
                          lambda i, j: (i, j, 0, 0, 0))
    return pl.pallas_call(
        _gather_body,
        grid=(b, h),
        in_specs=[spec_i, spec_k, spec_k],
        out_specs=[spec_o, spec_o],
        out_shape=[jax.ShapeDtypeStruct((b, h, nqb, SELC, BLKC * DHC),
                                        jnp.bfloat16)] * 2,
    )(idx, k2, v2)


_QG = 4


def _fine2_body(q_ref, fk_ref, fv_ref, oc_ref, g_ref, o_ref):
    s = q_ref.shape[2]
    nqb = s // BLKC
    ngrp = nqb // _QG
    rows = _QG * BLKC
    cols = _QG * SELC * BLKC
    scale = DHC ** -0.5
    r_blk = jax.lax.broadcasted_iota(jnp.int32, (rows, cols), 0) // BLKC
    c_blk = jax.lax.broadcasted_iota(jnp.int32, (rows, cols), 1) // (SELC * BLKC)
    mask = r_blk == c_blk
    for t in range(ngrp):
        qg = q_ref[0, 0, t * rows:(t + 1) * rows, :].astype(jnp.bfloat16)
        fkg = fk_ref[0, 0, t * cols:(t + 1) * cols, :]
        fvg = fv_ref[0, 0, t * cols:(t + 1) * cols, :]
        sg = jax.lax.dot_general(qg, fkg, (((1,), (1,)), ((), ())),
                                 preferred_element_type=jnp.float32) * scale
        sg = jnp.where(mask, sg, -1e30)
        m = jnp.max(sg, axis=1, keepdims=True)
        p = jnp.exp(sg - m)
        p = p / jnp.sum(p, axis=1, keepdims=True)
        og = jnp.dot(p.astype(jnp.bfloat16), fvg,
                     preferred_element_type=jnp.float32)
        gg = jax.nn.sigmoid(g_ref[0, 0, t * rows:(t + 1) * rows, :])
        oc = oc_ref[0, 0, t * rows:(t + 1) * rows, :]
        o_ref[0, 0, t * rows:(t + 1) * rows, :] = (
            gg[:, 0:1] * oc + gg[:, 1:2] * og)


def _fine2(q, fk3, fv3, oc, g):
    b, h, s, _ = q.shape
    spec_s = pl.BlockSpec((1, 1, s, DHC), lambda i, j: (i, j, 0, 0))
    spec_f = pl.BlockSpec((1, 1, SELC * s, DHC), lambda i, j: (i, j, 0, 0))
    spec_g = pl.BlockSpec((1, 1, s, 2), lambda i, j: (i, j, 0, 0))
    return pl.pallas_call(
        _fine2_body,
        grid=(b, h),
        in_specs=[spec_s, spec_f, spec_f, spec_s, spec_g],
        out_specs=spec_s,
        out_shape=jax.ShapeDtypeStruct((b, h, s, DHC), jnp.float32),
    )(q, fk3, fv3, oc, g)


def _fnorm_body(x_ref, nw_ref, o_ref):
    o_ref[...] = _rms(x_ref[...], nw_ref[...])


def _sparse_attn(x, p):
    b, s, _ = x.shape
    rows = b * s
    x2 = x.reshape(rows, DIMC)
    wcat = jnp.concatenate([p['wq'], p['wk'], p['wv'], p['wg']], axis=1)
    y = _rows_mm(_qkvg_body, x2, [p['norm_w'].reshape(1, DIMC), wcat],
                 3 * DIMC + 2 * HEADSC)
    q = y[:, :DIMC].reshape(b, s, HEADSC, DHC).transpose(0, 2, 1, 3)
    k = y[:, DIMC:2 * DIMC].reshape(b, s, HEADSC, DHC).transpose(0, 2, 1, 3)
    v = y[:, 2 * DIMC:3 * DIMC].reshape(b, s, HEADSC, DHC).transpose(0, 2, 1, 3)
    g = y[:, 3 * DIMC:].reshape(b, s, HEADSC, 2).transpose(0, 2, 1, 3)
    ck, cv = _pool(k, v, p['wp_k'], p['wp_v'])
    oc, idx = _coarse(q, ck, cv)
    nb = s // BLKC
    fk, fv = _gather(idx, k.reshape(b, HEADSC, nb, BLKC * DHC),
                     v.reshape(b, HEADSC, nb, BLKC * DHC))
    fk3 = fk.reshape(b, HEADSC, SELC * s, DHC)
    fv3 = fv.reshape(b, HEADSC, SELC * s, DHC)
    o = _fine2(q, fk3, fv3, oc, g)
    o2 = o.transpose(0, 2, 1, 3).reshape(rows, HEADSC * DHC)
    out = _rows_mm(_oproj_body, o2, [p['wo']], DIMC, extra_row=x2)
    return out.reshape(b, s, DIMC)


def _ffl(x, p):
    b, s, _ = x.shape
    rows = b * s
    x2 = x.reshape(rows, DIMC)
    h = _rows_mm(_ff1_body, x2,
                 [p['ffnorm_w'].reshape(1, DIMC), p['w1'], p['b1'].reshape(1, FFH)],
                 FFH)
    out = _rows_mm(_ff2_body, h, [p['w2'], p['b2'].reshape(1, DIMC)], DIMC,
                   extra_row=x2)
    return out.reshape(b, s, DIMC)


def kernel(tokens0, tokens1, tokens2, params):
    outs = []
    for i, tokens in enumerate((tokens0, tokens1, tokens2)):
        t = tokens
        for layer in params['levels'][i]:
            t = _sparse_attn(t, layer)
            t = _ffl(t, layer)
        b, s, _ = t.shape
        t2 = t.reshape(b * s, DIMC)
        o = _rows_mm(_fnorm_body, t2,
                     [params['final_norm'].reshape(1, DIMC)], DIMC)
        outs.append(o.reshape(b, s, DIMC))
    return tuple(outs)


# speedup vs baseline: 1.3299x; 1.2021x over previous
import jax
import jax.numpy as jnp
from jax.experimental import pallas as pl

DIMC = 1024
HEADSC = 16
DHC = 64
BLKC = 32
SELC = 8
FFH = 4096
ROWT = 256


def _rms(x, w):
    return x * jax.lax.rsqrt(jnp.mean(x * x, axis=-1, keepdims=True) + 1e-6) * w


def _qkvg_body(x_ref, nw_ref, w_ref, q_ref, k_ref, v_ref, g_ref):
    xn = _rms(x_ref[...], nw_ref[...])
    y = jnp.dot(xn, w_ref[...], preferred_element_type=jnp.float32)
    q_ref[...] = y[:, :DIMC]
    k_ref[...] = y[:, DIMC:2 * DIMC]
    v_ref[...] = y[:, 2 * DIMC:3 * DIMC]
    g_ref[...] = y[:, 3 * DIMC:]


def _qkvg(x2, norm_w, wcat):
    n = x2.shape[0]
    gcols = 2 * HEADSC
    return pl.pallas_call(
        _qkvg_body,
        grid=(n // ROWT,),
        in_specs=[pl.BlockSpec((ROWT, DIMC), lambda i: (i, 0)),
                  pl.BlockSpec((1, DIMC), lambda i: (0, 0)),
                  pl.BlockSpec((DIMC, 3 * DIMC + gcols), lambda i: (0, 0))],
        out_specs=[pl.BlockSpec((ROWT, DIMC), lambda i: (i, 0))] * 3
        + [pl.BlockSpec((ROWT, gcols), lambda i: (i, 0))],
        out_shape=[jax.ShapeDtypeStruct((n, DIMC), jnp.float32)] * 3
        + [jax.ShapeDtypeStruct((n, gcols), jnp.float32)],
    )(x2, norm_w, wcat)


def _oproj_body(a_ref, w_ref, r_ref, o_ref):
    o_ref[...] = jnp.dot(a_ref[...], w_ref[...],
                         preferred_element_type=jnp.float32) + r_ref[...]


def _ff1_body(x_ref, nw_ref, w1_ref, b1_ref, o_ref):
    xn = _rms(x_ref[...], nw_ref[...])
    h = jnp.dot(xn, w1_ref[...], preferred_element_type=jnp.float32) + b1_ref[...]
    o_ref[...] = jax.nn.gelu(h)


def _ff2_body(h_ref, w2_ref, b2_ref, r_ref, o_ref):
    o_ref[...] = (jnp.dot(h_ref[...], w2_ref[...],
                          preferred_element_type=jnp.float32)
                  + b2_ref[...] + r_ref[...])


def _rows_mm(body, x, aux_full, out_cols, extra_row=None):
    n, _ = x.shape
    in_specs = [pl.BlockSpec((ROWT, x.shape[1]), lambda i: (i, 0))]
    args = [x]
    for a in aux_full:
        in_specs.append(pl.BlockSpec(a.shape, lambda i, r=a.ndim: (0,) * r))
        args.append(a)
    if extra_row is not None:
        in_specs.append(pl.BlockSpec((ROWT, extra_row.shape[1]), lambda i: (i, 0)))
        args.append(extra_row)
    return pl.pallas_call(
        body,
        grid=(n // ROWT,),
        in_specs=in_specs,
        out_specs=pl.BlockSpec((ROWT, out_cols), lambda i: (i, 0)),
        out_shape=jax.ShapeDtypeStruct((n, out_cols), jnp.float32),
    )(*args)


def _pool_body(k_ref, v_ref, wk_ref, wv_ref, ck_ref, cv_ref, kt_ref, vt_ref):
    s = k_ref.shape[1]
    nb = s // BLKC
    for hh in range(2):
        for src, wp, dst, tout in (
                (k_ref, wk_ref, ck_ref, kt_ref),
                (v_ref, wv_ref, cv_ref, vt_ref)):
            xs = src[0, :, hh * DHC:(hh + 1) * DHC]
            tout[0, hh] = xs
            blocks = xs.reshape(nb, BLKC, DHC)
            logits = jax.lax.dot_general(
                xs, wp[...], (((1,), (0,)), ((), ())),
                preferred_element_type=jnp.float32).reshape(nb, BLKC, DHC)
            attn = jax.nn.softmax(logits, axis=-2)
            dst[0, hh] = jnp.sum(blocks * attn, axis=-2)


def _pool(k2, v2, wp_k, wp_v):
    b, s, _ = k2.shape
    nb = s // BLKC
    spec_in = pl.BlockSpec((1, s, 2 * DHC), lambda i, p: (i, 0, p))
    spec_w = pl.BlockSpec((DHC, DHC), lambda i, p: (0, 0))
    spec_c = pl.BlockSpec((1, 2, nb, DHC), lambda i, p: (i, p, 0, 0))
    spec_t = pl.BlockSpec((1, 2, s, DHC), lambda i, p: (i, p, 0, 0))
    return pl.pallas_call(
        _pool_body,
        grid=(b, HEADSC // 2),
        in_specs=[spec_in, spec_in, spec_w, spec_w],
        out_specs=[spec_c, spec_c, spec_t, spec_t],
        out_shape=[jax.ShapeDtypeStruct((b, HEADSC, nb, DHC), jnp.float32)] * 2
        + [jax.ShapeDtypeStruct((b, HEADSC, s, DHC), jnp.float32)] * 2,
    )(k2, v2, wp_k, wp_v)


def _coarse_body(q_ref, ck_ref, cv_ref, oc_ref, idx_ref):
    s = q_ref.shape[1]
    nb = ck_ref.shape[2]
    nqb = s // BLKC
    scale = DHC ** -0.5
    for hh in range(2):
        qs = q_ref[0, :, hh * DHC:(hh + 1) * DHC]
        sc = jax.lax.dot_general(qs, ck_ref[0, hh], (((1,), (1,)), ((), ())),
                                 preferred_element_type=jnp.float32) * scale
        attn = jax.nn.softmax(sc, axis=-1)
        oc_ref[0, hh] = jnp.dot(attn, cv_ref[0, hh],
                                preferred_element_type=jnp.float32)
        imp = jnp.mean(attn.reshape(nqb, BLKC, nb), axis=1)
        lane = jax.lax.broadcasted_iota(jnp.int32, (nqb, nb), 1)
        cols = []
        for _ in range(SELC):
            best = jnp.argmax(imp, axis=1).astype(jnp.int32)
            cols.append(best)
            imp = jnp.where(lane == best[:, None], -jnp.inf, imp)
        idx_ref[0, hh] = jnp.stack(cols, axis=1)


def _coarse(q2, ck, cv):
    b, s, _ = q2.shape
    nb = ck.shape[2]
    nqb = s // BLKC
    return pl.pallas_call(
        _coarse_body,
        grid=(b, HEADSC // 2),
        in_specs=[pl.BlockSpec((1, s, 2 * DHC), lambda i, p: (i, 0, p)),
                  pl.BlockSpec((1, 2, nb, DHC), lambda i, p: (i, p, 0, 0)),
                  pl.BlockSpec((1, 2, nb, DHC), lambda i, p: (i, p, 0, 0))],
        out_specs=[pl.BlockSpec((1, 2, s, DHC), lambda i, p: (i, p, 0, 0)),
                   pl.BlockSpec((1, 2, nqb, SELC), lambda i, p: (i, p, 0, 0))],
        out_shape=[jax.ShapeDtypeStruct((b, HEADSC, s, DHC), jnp.float32),
                   jax.ShapeDtypeStruct((b, HEADSC, nqb, SELC), jnp.int32)],
    )(q2, ck, cv)


def _gather_body(idx_ref, k2_ref, v2_ref, fk_ref, fv_ref):
    nb = k2_ref.shape[2]
    nqb = idx_ref.shape[2]
    idx = idx_ref[0, 0]
    k2b = k2_ref[0, 0].astype(jnp.bfloat16)
    v2b = v2_ref[0, 0].astype(jnp.bfloat16)
    idxcat = jnp.concatenate([idx[:, j:j + 1] for j in range(SELC)], axis=0)
    lane = jax.lax.broadcasted_iota(jnp.int32, (SELC * nqb, nb), 1)
    oh = (lane == idxcat).astype(jnp.bfloat16)
    fk_ref[0, 0] = jnp.dot(oh, k2b,
                           preferred_element_type=jnp.float32).astype(jnp.bfloat16)
    fv_ref[0, 0] = jnp.dot(oh, v2b,
                           preferred_element_type=jnp.float32).astype(jnp.bfloat16)


def _gather(idx, k2, v2):
    b, h, nb, _ = k2.shape
    nqb = idx.shape[2]
    spec_i = pl.BlockSpec((1, 1, nqb, SELC), lambda i, j: (i, j, 0, 0))
    spec_k = pl.BlockSpec((1, 1, nb, BLKC * DHC), lambda i, j: (i, j, 0, 0))
    spec_o = pl.BlockSpec((1, 1, SELC * nqb, BLKC * DHC),
                          lambda i, j: (i, j, 0, 0))
    return pl.pallas_call(
        _gather_body,
        grid=(b, h),
        in_specs=[spec_i, spec_k, spec_k],
        out_specs=[spec_o, spec_o],
        out_shape=[jax.ShapeDtypeStruct((b, h, SELC * nqb, BLKC * DHC),
                                        jnp.bfloat16)] * 2,
    )(idx, k2, v2)


_QG = 2


def _fine2_body(q_ref, fk_ref, fv_ref, oc_ref, g_ref, o_ref):
    s = q_ref.shape[1]
    nqb = s // BLKC
    ngrp = nqb // _QG
    rows = _QG * BLKC
    cols = _QG * SELC * BLKC
    scale = DHC ** -0.5
    r_blk = jax.lax.broadcasted_iota(jnp.int32, (rows, cols), 0) // BLKC
    c_blk = (jax.lax.broadcasted_iota(jnp.int32, (rows, cols), 1) // BLKC) % _QG
    bias = jnp.where(r_blk == c_blk, 0.0, -1e30).astype(jnp.float32)
    for hh in range(2):
        for t in range(ngrp):
            r0 = t * rows
            qg = (q_ref[0, r0:r0 + rows, hh * DHC:(hh + 1) * DHC]
                  * scale).astype(jnp.bfloat16)
            fkg = jnp.concatenate(
                [fk_ref[0, hh, j * s + r0:j * s + r0 + rows, :]
                 for j in range(SELC)], axis=0)
            fvg = jnp.concatenate(
                [fv_ref[0, hh, j * s + r0:j * s + r0 + rows, :]
                 for j in range(SELC)], axis=0)
            sg = jax.lax.dot_general(qg, fkg, (((1,), (1,)), ((), ())),
                                     preferred_element_type=jnp.float32) + bias
            m = jnp.max(sg, axis=1, keepdims=True)
            p = jnp.exp(sg - m)
            og = jnp.dot(p.astype(jnp.bfloat16), fvg,
                         preferred_element_type=jnp.float32)
            og = og / jnp.sum(p, axis=1, keepdims=True)
            gg = jax.nn.sigmoid(g_ref[0, hh, r0:r0 + rows, :])
            oc = oc_ref[0, hh, r0:r0 + rows, :]
            o_ref[0, r0:r0 + rows, hh * DHC:(hh + 1) * DHC] = (
                gg[:, 0:1] * oc + gg[:, 1:2] * og)


def _fine2(q2, fk3, fv3, oc, gt):
    b, s, _ = q2.shape
    return pl.pallas_call(
        _fine2_body,
        grid=(b, HEADSC // 2),
        in_specs=[pl.BlockSpec((1, s, 2 * DHC), lambda i, p: (i, 0, p)),
                  pl.BlockSpec((1, 2, SELC * s, DHC), lambda i, p: (i, p, 0, 0)),
                  pl.BlockSpec((1, 2, SELC * s, DHC), lambda i, p: (i, p, 0, 0)),
                  pl.BlockSpec((1, 2, s, DHC), lambda i, p: (i, p, 0, 0)),
                  pl.BlockSpec((1, 2, s, 2), lambda i, p: (i, p, 0, 0))],
        out_specs=pl.BlockSpec((1, s, 2 * DHC), lambda i, p: (i, 0, p)),
        out_shape=jax.ShapeDtypeStruct((b, s, HEADSC * DHC), jnp.float32),
    )(q2, fk3, fv3, oc, gt)


def _fnorm_body(x_ref, nw_ref, o_ref):
    o_ref[...] = _rms(x_ref[...], nw_ref[...])


def _sparse_attn(x, p):
    b, s, _ = x.shape
    rows = b * s
    nb = s // BLKC
    x2 = x.reshape(rows, DIMC)
    wcat = jnp.concatenate([p['wq'], p['wk'], p['wv'], p['wg']], axis=1)
    q, k, v, g = _qkvg(x2, p['norm_w'].reshape(1, DIMC), wcat)
    q2 = q.reshape(b, s, DIMC)
    k2 = k.reshape(b, s, DIMC)
    v2 = v.reshape(b, s, DIMC)
    gt = g.reshape(b, s, HEADSC, 2).transpose(0, 2, 1, 3)
    ck, cv, kt, vt = _pool(k2, v2, p['wp_k'], p['wp_v'])
    oc, idx = _coarse(q2, ck, cv)
    fk, fv = _gather(idx, kt.reshape(b, HEADSC, nb, BLKC * DHC),
                     vt.reshape(b, HEADSC, nb, BLKC * DHC))
    fk3 = fk.reshape(b, HEADSC, SELC * s, DHC)
    fv3 = fv.reshape(b, HEADSC, SELC * s, DHC)
    o2 = _fine2(q2, fk3, fv3, oc, gt)
    out = _rows_mm(_oproj_body, o2.reshape(rows, DIMC), [p['wo']], DIMC,
                   extra_row=x2)
    return out.reshape(b, s, DIMC)


def _ffl(x, p):
    b, s, _ = x.shape
    rows = b * s
    x2 = x.reshape(rows, DIMC)
    h = _rows_mm(_ff1_body, x2,
                 [p['ffnorm_w'].reshape(1, DIMC), p['w1'], p['b1'].reshape(1, FFH)],
                 FFH)
    out = _rows_mm(_ff2_body, h, [p['w2'], p['b2'].reshape(1, DIMC)], DIMC,
                   extra_row=x2)
    return out.reshape(b, s, DIMC)


def kernel(tokens0, tokens1, tokens2, params):
    outs = []
    for i, tokens in enumerate((tokens0, tokens1, tokens2)):
        t = tokens
        for layer in params['levels'][i]:
            t = _sparse_attn(t, layer)
            t = _ffl(t, layer)
        b, s, _ = t.shape
        t2 = t.reshape(b * s, DIMC)
        o = _rows_mm(_fnorm_body, t2,
                     [params['final_norm'].reshape(1, DIMC)], DIMC)
        outs.append(o.reshape(b, s, DIMC))
    return tuple(outs)


# speedup vs baseline: 1.5480x; 1.1640x over previous
import jax
import jax.numpy as jnp
from jax.experimental import pallas as pl

DIMC = 1024
HEADSC = 16
DHC = 64
BLKC = 32
SELC = 8
FFH = 4096
ROWT = 256


def _rms(x, w):
    return x * jax.lax.rsqrt(jnp.mean(x * x, axis=-1, keepdims=True) + 1e-6) * w


def _qkvg_body(x_ref, nw_ref, w_ref, q_ref, k_ref, v_ref, g_ref):
    xn = _rms(x_ref[...], nw_ref[...])
    y = jnp.dot(xn, w_ref[...], preferred_element_type=jnp.float32)
    q_ref[...] = y[:, :DIMC]
    k_ref[...] = y[:, DIMC:2 * DIMC]
    v_ref[...] = y[:, 2 * DIMC:3 * DIMC]
    g_ref[...] = y[:, 3 * DIMC:]


def _qkvg(x2, norm_w, wcat):
    n = x2.shape[0]
    gcols = 2 * HEADSC
    return pl.pallas_call(
        _qkvg_body,
        grid=(n // ROWT,),
        in_specs=[pl.BlockSpec((ROWT, DIMC), lambda i: (i, 0)),
                  pl.BlockSpec((1, DIMC), lambda i: (0, 0)),
                  pl.BlockSpec((DIMC, 3 * DIMC + gcols), lambda i: (0, 0))],
        out_specs=[pl.BlockSpec((ROWT, DIMC), lambda i: (i, 0))] * 3
        + [pl.BlockSpec((ROWT, gcols), lambda i: (i, 0))],
        out_shape=[jax.ShapeDtypeStruct((n, DIMC), jnp.float32)] * 3
        + [jax.ShapeDtypeStruct((n, gcols), jnp.float32)],
    )(x2, norm_w, wcat)


def _oproj_body(a_ref, w_ref, r_ref, o_ref):
    o_ref[...] = jnp.dot(a_ref[...], w_ref[...],
                         preferred_element_type=jnp.float32) + r_ref[...]


def _ff1_body(x_ref, nw_ref, w1_ref, b1_ref, o_ref):
    xn = _rms(x_ref[...], nw_ref[...])
    h = jnp.dot(xn, w1_ref[...], preferred_element_type=jnp.float32) + b1_ref[...]
    o_ref[...] = jax.nn.gelu(h)


def _ff2_body(h_ref, w2_ref, b2_ref, r_ref, o_ref):
    o_ref[...] = (jnp.dot(h_ref[...], w2_ref[...],
                          preferred_element_type=jnp.float32)
                  + b2_ref[...] + r_ref[...])


def _rows_mm(body, x, aux_full, out_cols, extra_row=None):
    n, _ = x.shape
    in_specs = [pl.BlockSpec((ROWT, x.shape[1]), lambda i: (i, 0))]
    args = [x]
    for a in aux_full:
        in_specs.append(pl.BlockSpec(a.shape, lambda i, r=a.ndim: (0,) * r))
        args.append(a)
    if extra_row is not None:
        in_specs.append(pl.BlockSpec((ROWT, extra_row.shape[1]), lambda i: (i, 0)))
        args.append(extra_row)
    return pl.pallas_call(
        body,
        grid=(n // ROWT,),
        in_specs=in_specs,
        out_specs=pl.BlockSpec((ROWT, out_cols), lambda i: (i, 0)),
        out_shape=jax.ShapeDtypeStruct((n, out_cols), jnp.float32),
    )(*args)


def _pool_body(k_ref, v_ref, wk_ref, wv_ref, ck_ref, cv_ref, kt_ref, vt_ref):
    s = k_ref.shape[1]
    nb = s // BLKC
    for hh in range(2):
        for src, wp, dst, tout in (
                (k_ref, wk_ref, ck_ref, kt_ref),
                (v_ref, wv_ref, cv_ref, vt_ref)):
            xs = src[0, :, hh * DHC:(hh + 1) * DHC]
            tout[0, hh] = xs
            blocks = xs.reshape(nb, BLKC, DHC)
            logits = jax.lax.dot_general(
                xs, wp[...], (((1,), (0,)), ((), ())),
                preferred_element_type=jnp.float32).reshape(nb, BLKC, DHC)
            attn = jax.nn.softmax(logits, axis=-2)
            dst[0, hh] = jnp.sum(blocks * attn, axis=-2)


def _pool(k2, v2, wp_k, wp_v):
    b, s, _ = k2.shape
    nb = s // BLKC
    spec_in = pl.BlockSpec((1, s, 2 * DHC), lambda i, p: (i, 0, p))
    spec_w = pl.BlockSpec((DHC, DHC), lambda i, p: (0, 0))
    spec_c = pl.BlockSpec((1, 2, nb, DHC), lambda i, p: (i, p, 0, 0))
    spec_t = pl.BlockSpec((1, 2, s, DHC), lambda i, p: (i, p, 0, 0))
    return pl.pallas_call(
        _pool_body,
        grid=(b, HEADSC // 2),
        in_specs=[spec_in, spec_in, spec_w, spec_w],
        out_specs=[spec_c, spec_c, spec_t, spec_t],
        out_shape=[jax.ShapeDtypeStruct((b, HEADSC, nb, DHC), jnp.float32)] * 2
        + [jax.ShapeDtypeStruct((b, HEADSC, s, DHC), jnp.float32)] * 2,
    )(k2, v2, wp_k, wp_v)


def _coarse_body(q_ref, ck_ref, cv_ref, oc_ref, idx_ref):
    s = q_ref.shape[1]
    nb = ck_ref.shape[2]
    nqb = s // BLKC
    scale = DHC ** -0.5
    for hh in range(2):
        qs = q_ref[0, :, hh * DHC:(hh + 1) * DHC]
        sc = jax.lax.dot_general(qs, ck_ref[0, hh], (((1,), (1,)), ((), ())),
                                 preferred_element_type=jnp.float32) * scale
        attn = jax.nn.softmax(sc, axis=-1)
        oc_ref[0, hh] = jnp.dot(attn, cv_ref[0, hh],
                                preferred_element_type=jnp.float32)
        imp = jnp.mean(attn.reshape(nqb, BLKC, nb), axis=1)
        lane = jax.lax.broadcasted_iota(jnp.int32, (nqb, nb), 1)
        cols = []
        for _ in range(SELC):
            best = jnp.argmax(imp, axis=1).astype(jnp.int32)
            cols.append(best)
            imp = jnp.where(lane == best[:, None], -jnp.inf, imp)
        idx_ref[0, hh] = jnp.stack(cols, axis=1)


def _coarse(q2, ck, cv):
    b, s, _ = q2.shape
    nb = ck.shape[2]
    nqb = s // BLKC
    return pl.pallas_call(
        _coarse_body,
        grid=(b, HEADSC // 2),
        in_specs=[pl.BlockSpec((1, s, 2 * DHC), lambda i, p: (i, 0, p)),
                  pl.BlockSpec((1, 2, nb, DHC), lambda i, p: (i, p, 0, 0)),
                  pl.BlockSpec((1, 2, nb, DHC), lambda i, p: (i, p, 0, 0))],
        out_specs=[pl.BlockSpec((1, 2, s, DHC), lambda i, p: (i, p, 0, 0)),
                   pl.BlockSpec((1, 2, nqb, SELC), lambda i, p: (i, p, 0, 0))],
        out_shape=[jax.ShapeDtypeStruct((b, HEADSC, s, DHC), jnp.float32),
                   jax.ShapeDtypeStruct((b, HEADSC, nqb, SELC), jnp.int32)],
    )(q2, ck, cv)


def _gather_body(idx_ref, k2_ref, v2_ref, fk_ref, fv_ref):
    nb = k2_ref.shape[2]
    nqb = idx_ref.shape[2]
    idx = idx_ref[0, 0]
    k2b = k2_ref[0, 0].astype(jnp.bfloat16)
    v2b = v2_ref[0, 0].astype(jnp.bfloat16)
    idxcat = jnp.concatenate([idx[:, j:j + 1] for j in range(SELC)], axis=0)
    lane = jax.lax.broadcasted_iota(jnp.int32, (SELC * nqb, nb), 1)
    oh = (lane == idxcat).astype(jnp.bfloat16)
    fk_ref[0, 0] = jnp.dot(oh, k2b,
                           preferred_element_type=jnp.float32).astype(jnp.bfloat16)
    fv_ref[0, 0] = jnp.dot(oh, v2b,
                           preferred_element_type=jnp.float32).astype(jnp.bfloat16)


def _gather(idx, k2, v2):
    b, h, nb, _ = k2.shape
    nqb = idx.shape[2]
    spec_i = pl.BlockSpec((1, 1, nqb, SELC), lambda i, j: (i, j, 0, 0))
    spec_k = pl.BlockSpec((1, 1, nb, BLKC * DHC), lambda i, j: (i, j, 0, 0))
    spec_o = pl.BlockSpec((1, 1, SELC * nqb, BLKC * DHC),
                          lambda i, j: (i, j, 0, 0))
    return pl.pallas_call(
        _gather_body,
        grid=(b, h),
        in_specs=[spec_i, spec_k, spec_k],
        out_specs=[spec_o, spec_o],
        out_shape=[jax.ShapeDtypeStruct((b, h, SELC * nqb, BLKC * DHC),
                                        jnp.bfloat16)] * 2,
    )(idx, k2, v2)


_QG = 2


def _fine2_body(q_ref, fk_ref, fv_ref, oc_ref, g_ref, o_ref):
    s = q_ref.shape[1]
    half = s // 2
    nqb = s // BLKC
    ngrp = nqb // _QG
    rows = _QG * BLKC
    pcols = _QG * SELC * BLKC // 2
    psl = _QG * BLKC // 2
    scale = DHC ** -0.5
    zpad = jnp.zeros((rows, DHC), jnp.bfloat16)
    r_blk = jax.lax.broadcasted_iota(jnp.int32, (rows, pcols), 0) // BLKC
    c_blk = (jax.lax.broadcasted_iota(jnp.int32, (rows, pcols), 1)
             // (BLKC // 2)) % _QG
    bias = jnp.where(r_blk == c_blk, 0.0, -1e30).astype(jnp.float32)
    dn = (((1,), (1,)), ((), ()))
    for hh in range(2):
        for t in range(ngrp):
            r0 = t * rows
            qg = (q_ref[0, r0:r0 + rows, hh * DHC:(hh + 1) * DHC]
                  * scale).astype(jnp.bfloat16)
            qe = jnp.concatenate([qg, zpad], axis=1)
            qo = jnp.concatenate([zpad, qg], axis=1)
            fkg = jnp.concatenate(
                [fk_ref[0, hh, j * half + t * psl:j * half + (t + 1) * psl, :]
                 for j in range(SELC)], axis=0)
            fvg = jnp.concatenate(
                [fv_ref[0, hh, j * half + t * psl:j * half + (t + 1) * psl, :]
                 for j in range(SELC)], axis=0)
            se = jax.lax.dot_general(qe, fkg, dn,
                                     preferred_element_type=jnp.float32) + bias
            so = jax.lax.dot_general(qo, fkg, dn,
                                     preferred_element_type=jnp.float32) + bias
            m = jnp.maximum(jnp.max(se, axis=1, keepdims=True),
                            jnp.max(so, axis=1, keepdims=True))
            pe = jnp.exp(se - m)
            po = jnp.exp(so - m)
            d = (jnp.sum(pe, axis=1, keepdims=True)
                 + jnp.sum(po, axis=1, keepdims=True))
            oge = jnp.dot(pe.astype(jnp.bfloat16), fvg,
                          preferred_element_type=jnp.float32)
            ogo = jnp.dot(po.astype(jnp.bfloat16), fvg,
                          preferred_element_type=jnp.float32)
            og = (oge[:, :DHC] + ogo[:, DHC:]) / d
            gg = jax.nn.sigmoid(g_ref[0, hh, r0:r0 + rows, :])
            oc = oc_ref[0, hh, r0:r0 + rows, :]
            o_ref[0, r0:r0 + rows, hh * DHC:(hh + 1) * DHC] = (
                gg[:, 0:1] * oc + gg[:, 1:2] * og)


def _fine2(q2, fkp, fvp, oc, gt):
    b, s, _ = q2.shape
    return pl.pallas_call(
        _fine2_body,
        grid=(b, HEADSC // 2),
        in_specs=[pl.BlockSpec((1, s, 2 * DHC), lambda i, p: (i, 0, p)),
                  pl.BlockSpec((1, 2, SELC * s // 2, 2 * DHC),
                               lambda i, p: (i, p, 0, 0)),
                  pl.BlockSpec((1, 2, SELC * s // 2, 2 * DHC),
                               lambda i, p: (i, p, 0, 0)),
                  pl.BlockSpec((1, 2, s, DHC), lambda i, p: (i, p, 0, 0)),
                  pl.BlockSpec((1, 2, s, 2), lambda i, p: (i, p, 0, 0))],
        out_specs=pl.BlockSpec((1, s, 2 * DHC), lambda i, p: (i, 0, p)),
        out_shape=jax.ShapeDtypeStruct((b, s, HEADSC * DHC), jnp.float32),
    )(q2, fkp, fvp, oc, gt)


def _fnorm_body(x_ref, nw_ref, o_ref):
    o_ref[...] = _rms(x_ref[...], nw_ref[...])


def _sparse_attn(x, p):
    b, s, _ = x.shape
    rows = b * s
    nb = s // BLKC
    x2 = x.reshape(rows, DIMC)
    wcat = jnp.concatenate([p['wq'], p['wk'], p['wv'], p['wg']], axis=1)
    q, k, v, g = _qkvg(x2, p['norm_w'].reshape(1, DIMC), wcat)
    q2 = q.reshape(b, s, DIMC)
    k2 = k.reshape(b, s, DIMC)
    v2 = v.reshape(b, s, DIMC)
    gt = g.reshape(b, s, HEADSC, 2).transpose(0, 2, 1, 3)
    ck, cv, kt, vt = _pool(k2, v2, p['wp_k'], p['wp_v'])
    oc, idx = _coarse(q2, ck, cv)
    fk, fv = _gather(idx, kt.reshape(b, HEADSC, nb, BLKC * DHC),
                     vt.reshape(b, HEADSC, nb, BLKC * DHC))
    fkp = fk.reshape(b, HEADSC, SELC * s // 2, 2 * DHC)
    fvp = fv.reshape(b, HEADSC, SELC * s // 2, 2 * DHC)
    o2 = _fine2(q2, fkp, fvp, oc, gt)
    out = _rows_mm(_oproj_body, o2.reshape(rows, DIMC), [p['wo']], DIMC,
                   extra_row=x2)
    return out.reshape(b, s, DIMC)


def _ffl(x, p):
    b, s, _ = x.shape
    rows = b * s
    x2 = x.reshape(rows, DIMC)
    h = _rows_mm(_ff1_body, x2,
                 [p['ffnorm_w'].reshape(1, DIMC), p['w1'], p['b1'].reshape(1, FFH)],
                 FFH)
    out = _rows_mm(_ff2_body, h, [p['w2'], p['b2'].reshape(1, DIMC)], DIMC,
                   extra_row=x2)
    return out.reshape(b, s, DIMC)


def kernel(tokens0, tokens1, tokens2, params):
    outs = []
    for i, tokens in enumerate((tokens0, tokens1, tokens2)):
        t = tokens
        for layer in params['levels'][i]:
            t = _sparse_attn(t, layer)
            t = _ffl(t, layer)
        b, s, _ = t.shape
        t2 = t.reshape(b * s, DIMC)
        o = _rows_mm(_fnorm_body, t2,
                     [params['final_norm'].reshape(1, DIMC)], DIMC)
        outs.append(o.reshape(b, s, DIMC))
    return tuple(outs)


# speedup vs baseline: 1.5788x; 1.0199x over previous
import jax
import jax.numpy as jnp
from jax.experimental import pallas as pl

DIMC = 1024
HEADSC = 16
DHC = 64
BLKC = 32
SELC = 8
FFH = 4096
ROWT = 256


def _rms(x, w):
    return x * jax.lax.rsqrt(jnp.mean(x * x, axis=-1, keepdims=True) + 1e-6) * w


def _qkvg_body(x_ref, nw_ref, w_ref, q_ref, k_ref, v_ref, g_ref):
    xn = _rms(x_ref[...], nw_ref[...])
    y = jnp.dot(xn, w_ref[...], preferred_element_type=jnp.float32)
    q_ref[...] = y[:, :DIMC]
    k_ref[...] = y[:, DIMC:2 * DIMC]
    v_ref[...] = y[:, 2 * DIMC:3 * DIMC]
    g_ref[...] = y[:, 3 * DIMC:]


def _qkvg(x2, norm_w, wcat):
    n = x2.shape[0]
    gcols = 2 * HEADSC
    return pl.pallas_call(
        _qkvg_body,
        grid=(n // ROWT,),
        in_specs=[pl.BlockSpec((ROWT, DIMC), lambda i: (i, 0)),
                  pl.BlockSpec((1, DIMC), lambda i: (0, 0)),
                  pl.BlockSpec((DIMC, 3 * DIMC + gcols), lambda i: (0, 0))],
        out_specs=[pl.BlockSpec((ROWT, DIMC), lambda i: (i, 0))] * 3
        + [pl.BlockSpec((ROWT, gcols), lambda i: (i, 0))],
        out_shape=[jax.ShapeDtypeStruct((n, DIMC), jnp.float32)] * 3
        + [jax.ShapeDtypeStruct((n, gcols), jnp.float32)],
    )(x2, norm_w, wcat)


def _oproj_body(a_ref, w_ref, r_ref, o_ref):
    o_ref[...] = jnp.dot(a_ref[...], w_ref[...],
                         preferred_element_type=jnp.float32) + r_ref[...]


def _ff1_body(x_ref, nw_ref, w1_ref, b1_ref, o_ref):
    xn = _rms(x_ref[...], nw_ref[...])
    h = jnp.dot(xn, w1_ref[...], preferred_element_type=jnp.float32) + b1_ref[...]
    o_ref[...] = jax.nn.gelu(h)


def _ff2_body(h_ref, w2_ref, b2_ref, r_ref, o_ref):
    o_ref[...] = (jnp.dot(h_ref[...], w2_ref[...],
                          preferred_element_type=jnp.float32)
                  + b2_ref[...] + r_ref[...])


def _rows_mm(body, x, aux_full, out_cols, extra_row=None):
    n, _ = x.shape
    in_specs = [pl.BlockSpec((ROWT, x.shape[1]), lambda i: (i, 0))]
    args = [x]
    for a in aux_full:
        in_specs.append(pl.BlockSpec(a.shape, lambda i, r=a.ndim: (0,) * r))
        args.append(a)
    if extra_row is not None:
        in_specs.append(pl.BlockSpec((ROWT, extra_row.shape[1]), lambda i: (i, 0)))
        args.append(extra_row)
    return pl.pallas_call(
        body,
        grid=(n // ROWT,),
        in_specs=in_specs,
        out_specs=pl.BlockSpec((ROWT, out_cols), lambda i: (i, 0)),
        out_shape=jax.ShapeDtypeStruct((n, out_cols), jnp.float32),
    )(*args)


def _pool_body(k_ref, v_ref, wk_ref, wv_ref, ck_ref, cv_ref, kt_ref, vt_ref):
    s = k_ref.shape[1]
    nb = s // BLKC
    for hh in range(2):
        for src, wp, dst, tout in (
                (k_ref, wk_ref, ck_ref, kt_ref),
                (v_ref, wv_ref, cv_ref, vt_ref)):
            xs = src[0, :, hh * DHC:(hh + 1) * DHC]
            tout[0, hh] = xs.astype(jnp.bfloat16)
            blocks = xs.reshape(nb, BLKC, DHC)
            logits = jax.lax.dot_general(
                xs, wp[...], (((1,), (0,)), ((), ())),
                preferred_element_type=jnp.float32).reshape(nb, BLKC, DHC)
            attn = jax.nn.softmax(logits, axis=-2)
            dst[0, hh] = jnp.sum(blocks * attn, axis=-2)


def _pool(k2, v2, wp_k, wp_v):
    b, s, _ = k2.shape
    nb = s // BLKC
    spec_in = pl.BlockSpec((1, s, 2 * DHC), lambda i, p: (i, 0, p))
    spec_w = pl.BlockSpec((DHC, DHC), lambda i, p: (0, 0))
    spec_c = pl.BlockSpec((1, 2, nb, DHC), lambda i, p: (i, p, 0, 0))
    spec_t = pl.BlockSpec((1, 2, s, DHC), lambda i, p: (i, p, 0, 0))
    return pl.pallas_call(
        _pool_body,
        grid=(b, HEADSC // 2),
        in_specs=[spec_in, spec_in, spec_w, spec_w],
        out_specs=[spec_c, spec_c, spec_t, spec_t],
        out_shape=[jax.ShapeDtypeStruct((b, HEADSC, nb, DHC), jnp.float32)] * 2
        + [jax.ShapeDtypeStruct((b, HEADSC, s, DHC), jnp.bfloat16)] * 2,
    )(k2, v2, wp_k, wp_v)


def _coarse_body(q_ref, ck_ref, cv_ref, oc_ref, idx_ref):
    s = q_ref.shape[1]
    nb = ck_ref.shape[2]
    nqb = s // BLKC
    scale = DHC ** -0.5
    for hh in range(2):
        qs = q_ref[0, :, hh * DHC:(hh + 1) * DHC]
        sc = jax.lax.dot_general(qs, ck_ref[0, hh], (((1,), (1,)), ((), ())),
                                 preferred_element_type=jnp.float32) * scale
        attn = jax.nn.softmax(sc, axis=-1)
        oc_ref[0, hh] = jnp.dot(attn, cv_ref[0, hh],
                                preferred_element_type=jnp.float32)
        imp = jnp.mean(attn.reshape(nqb, BLKC, nb), axis=1)
        lane = jax.lax.broadcasted_iota(jnp.int32, (nqb, nb), 1)
        cols = []
        for _ in range(SELC):
            best = jnp.argmax(imp, axis=1).astype(jnp.int32)
            cols.append(best)
            imp = jnp.where(lane == best[:, None], -jnp.inf, imp)
        idx_ref[0, hh] = jnp.stack(cols, axis=1)


def _coarse(q2, ck, cv):
    b, s, _ = q2.shape
    nb = ck.shape[2]
    nqb = s // BLKC
    return pl.pallas_call(
        _coarse_body,
        grid=(b, HEADSC // 2),
        in_specs=[pl.BlockSpec((1, s, 2 * DHC), lambda i, p: (i, 0, p)),
                  pl.BlockSpec((1, 2, nb, DHC), lambda i, p: (i, p, 0, 0)),
                  pl.BlockSpec((1, 2, nb, DHC), lambda i, p: (i, p, 0, 0))],
        out_specs=[pl.BlockSpec((1, 2, s, DHC), lambda i, p: (i, p, 0, 0)),
                   pl.BlockSpec((1, 2, nqb, SELC), lambda i, p: (i, p, 0, 0))],
        out_shape=[jax.ShapeDtypeStruct((b, HEADSC, s, DHC), jnp.float32),
                   jax.ShapeDtypeStruct((b, HEADSC, nqb, SELC), jnp.int32)],
    )(q2, ck, cv)


def _gather_body(idx_ref, k2_ref, v2_ref, fk_ref, fv_ref):
    nb = k2_ref.shape[2]
    nqb = idx_ref.shape[2]
    idx = idx_ref[0, 0]
    k2b = k2_ref[0, 0]
    v2b = v2_ref[0, 0]
    idxcat = jnp.concatenate([idx[:, j:j + 1] for j in range(SELC)], axis=0)
    lane = jax.lax.broadcasted_iota(jnp.int32, (SELC * nqb, nb), 1)
    oh = (lane == idxcat).astype(jnp.bfloat16)
    fk_ref[0, 0] = jnp.dot(oh, k2b,
                           preferred_element_type=jnp.float32).astype(jnp.bfloat16)
    fv_ref[0, 0] = jnp.dot(oh, v2b,
                           preferred_element_type=jnp.float32).astype(jnp.bfloat16)


def _gather(idx, k2, v2):
    b, h, nb, _ = k2.shape
    nqb = idx.shape[2]
    spec_i = pl.BlockSpec((1, 1, nqb, SELC), lambda i, j: (i, j, 0, 0))
    spec_k = pl.BlockSpec((1, 1, nb, BLKC * DHC), lambda i, j: (i, j, 0, 0))
    spec_o = pl.BlockSpec((1, 1, SELC * nqb, BLKC * DHC),
                          lambda i, j: (i, j, 0, 0))
    return pl.pallas_call(
        _gather_body,
        grid=(b, h),
        in_specs=[spec_i, spec_k, spec_k],
        out_specs=[spec_o, spec_o],
        out_shape=[jax.ShapeDtypeStruct((b, h, SELC * nqb, BLKC * DHC),
                                        jnp.bfloat16)] * 2,
    )(idx, k2, v2)


_QG = 2


def _fine2_body(q_ref, fk_ref, fv_ref, oc_ref, g_ref, o_ref):
    s = q_ref.shape[1]
    half = s // 2
    nqb = s // BLKC
    ngrp = nqb // _QG
    rows = _QG * BLKC
    pcols = _QG * SELC * BLKC // 2
    psl = _QG * BLKC // 2
    scale = DHC ** -0.5
    zpad = jnp.zeros((rows, DHC), jnp.bfloat16)
    r_blk = jax.lax.broadcasted_iota(jnp.int32, (rows, pcols), 0) // BLKC
    c_blk = (jax.lax.broadcasted_iota(jnp.int32, (rows, pcols), 1)
             // (BLKC // 2)) % _QG
    bias = jnp.where(r_blk == c_blk, 0.0, -1e30).astype(jnp.float32)
    dn = (((1,), (1,)), ((), ()))
    for hh in range(2):
        for t in range(ngrp):
            r0 = t * rows
            qg = (q_ref[0, r0:r0 + rows, hh * DHC:(hh + 1) * DHC]
                  * scale).astype(jnp.bfloat16)
            qe = jnp.concatenate([qg, zpad], axis=1)
            qo = jnp.concatenate([zpad, qg], axis=1)
            fkg = jnp.concatenate(
                [fk_ref[0, hh, j * half + t * psl:j * half + (t + 1) * psl, :]
                 for j in range(SELC)], axis=0)
            fvg = jnp.concatenate(
                [fv_ref[0, hh, j * half + t * psl:j * half + (t + 1) * psl, :]
                 for j in range(SELC)], axis=0)
            se = jax.lax.dot_general(qe, fkg, dn,
                                     preferred_element_type=jnp.float32) + bias
            so = jax.lax.dot_general(qo, fkg, dn,
                                     preferred_element_type=jnp.float32) + bias
            m = jnp.maximum(jnp.max(se, axis=1, keepdims=True),
                            jnp.max(so, axis=1, keepdims=True))
            pe = jnp.exp(se - m)
            po = jnp.exp(so - m)
            d = (jnp.sum(pe, axis=1, keepdims=True)
                 + jnp.sum(po, axis=1, keepdims=True))
            oge = jnp.dot(pe.astype(jnp.bfloat16), fvg,
                          preferred_element_type=jnp.float32)
            ogo = jnp.dot(po.astype(jnp.bfloat16), fvg,
                          preferred_element_type=jnp.float32)
            og = (oge[:, :DHC] + ogo[:, DHC:]) / d
            gg = jax.nn.sigmoid(g_ref[0, hh, r0:r0 + rows, :])
            oc = oc_ref[0, hh, r0:r0 + rows, :]
            o_ref[0, r0:r0 + rows, hh * DHC:(hh + 1) * DHC] = (
                gg[:, 0:1] * oc + gg[:, 1:2] * og)


def _fine2(q2, fkp, fvp, oc, gt):
    b, s, _ = q2.shape
    return pl.pallas_call(
        _fine2_body,
        grid=(b, HEADSC // 2),
        in_specs=[pl.BlockSpec((1, s, 2 * DHC), lambda i, p: (i, 0, p)),
                  pl.BlockSpec((1, 2, SELC * s // 2, 2 * DHC),
                               lambda i, p: (i, p, 0, 0)),
                  pl.BlockSpec((1, 2, SELC * s // 2, 2 * DHC),
                               lambda i, p: (i, p, 0, 0)),
                  pl.BlockSpec((1, 2, s, DHC), lambda i, p: (i, p, 0, 0)),
                  pl.BlockSpec((1, 2, s, 2), lambda i, p: (i, p, 0, 0))],
        out_specs=pl.BlockSpec((1, s, 2 * DHC), lambda i, p: (i, 0, p)),
        out_shape=jax.ShapeDtypeStruct((b, s, HEADSC * DHC), jnp.float32),
    )(q2, fkp, fvp, oc, gt)


def _fnorm_body(x_ref, nw_ref, o_ref):
    o_ref[...] = _rms(x_ref[...], nw_ref[...])


def _sparse_attn(x, p):
    b, s, _ = x.shape
    rows = b * s
    nb = s // BLKC
    x2 = x.reshape(rows, DIMC)
    wcat = jnp.concatenate([p['wq'], p['wk'], p['wv'], p['wg']], axis=1)
    q, k, v, g = _qkvg(x2, p['norm_w'].reshape(1, DIMC), wcat)
    q2 = q.reshape(b, s, DIMC)
    k2 = k.reshape(b, s, DIMC)
    v2 = v.reshape(b, s, DIMC)
    gt = g.reshape(b, s, HEADSC, 2).transpose(0, 2, 1, 3)
    ck, cv, kt, vt = _pool(k2, v2, p['wp_k'], p['wp_v'])
    oc, idx = _coarse(q2, ck, cv)
    fk, fv = _gather(idx, kt.reshape(b, HEADSC, nb, BLKC * DHC),
                     vt.reshape(b, HEADSC, nb, BLKC * DHC))
    fkp = fk.reshape(b, HEADSC, SELC * s // 2, 2 * DHC)
    fvp = fv.reshape(b, HEADSC, SELC * s // 2, 2 * DHC)
    o2 = _fine2(q2, fkp, fvp, oc, gt)
    out = _rows_mm(_oproj_body, o2.reshape(rows, DIMC), [p['wo']], DIMC,
                   extra_row=x2)
    return out.reshape(b, s, DIMC)


def _ffl(x, p):
    b, s, _ = x.shape
    rows = b * s
    x2 = x.reshape(rows, DIMC)
    h = _rows_mm(_ff1_body, x2,
                 [p['ffnorm_w'].reshape(1, DIMC), p['w1'], p['b1'].reshape(1, FFH)],
                 FFH)
    out = _rows_mm(_ff2_body, h, [p['w2'], p['b2'].reshape(1, DIMC)], DIMC,
                   extra_row=x2)
    return out.reshape(b, s, DIMC)


def kernel(tokens0, tokens1, tokens2, params):
    outs = []
    for i, tokens in enumerate((tokens0, tokens1, tokens2)):
        t = tokens
        for layer in params['levels'][i]:
            t = _sparse_attn(t, layer)
            t = _ffl(t, layer)
        b, s, _ = t.shape
        t2 = t.reshape(b * s, DIMC)
        o = _rows_mm(_fnorm_body, t2,
                     [params['final_norm'].reshape(1, DIMC)], DIMC)
        outs.append(o.reshape(b, s, DIMC))
    return tuple(outs)
